# baseline (speedup 1.0000x reference)
import functools

import jax
import jax.numpy as jnp
from jax import lax
from jax.experimental import pallas as pl
from jax.experimental.pallas import tpu as pltpu
from jax.experimental.pallas import tpu_sc as plsc

F = 26
OUT = 16
B = 16384
NW = 32
BPT = B // NW
C = 128
NCHUNK = BPT // C
HI = F - 16
GR = 4
NG = C // GR


def _field_linear_sc(x, weight, off2, bias):
    mesh = plsc.VectorSubcoreMesh(core_axis_name="c", subcore_axis_name="s")

    @functools.partial(
        pl.kernel,
        out_type=jax.ShapeDtypeStruct((B, OUT), jnp.float32),
        mesh=mesh,
        compiler_params=pltpu.CompilerParams(use_tc_tiling_on_sc=False),
        scratch_types=[
            pltpu.VMEM((2, OUT), jnp.int32),
            pltpu.VMEM((OUT,), jnp.float32),
            pltpu.VMEM((2, C, F), jnp.int32),
            pltpu.VMEM((2, NG, GR * F), jnp.int32),
            pltpu.VMEM((2, C * F, OUT), jnp.float32),
            pltpu.VMEM((C, OUT), jnp.float32),
            pltpu.SemaphoreType.DMA,
            pltpu.SemaphoreType.DMA,
        ],
    )
    def k(x_hbm, w_hbm, off2_hbm, bias_hbm, out_hbm,
          off2_v, bias_v, xv, idx_v, gbuf, outb, sem0, sem1):
        cid = lax.axis_index("c")
        sid = lax.axis_index("s")
        wid = sid * 2 + cid
        tbase = wid * BPT
        sems = (sem0, sem1)

        pltpu.sync_copy(off2_hbm, off2_v)
        pltpu.sync_copy(bias_hbm, bias_v)
        bias_vec = bias_v[:]

        def stage_in(ci, pb):
            base = tbase + ci * C
            pltpu.sync_copy(x_hbm.at[pl.ds(base, C), :], xv.at[pb])
            off_lo = off2_v[0, :]
            off_hi = off2_v[1, :]

            def grp_body(g, carry):
                for r in range(GR):
                    j = g * GR + r
                    idx_v[pb, g, pl.ds(r * F, 16)] = (
                        xv[pb, j, pl.ds(0, 16)] + off_lo)
                    idx_v[pb, g, pl.ds(r * F + HI, 16)] = (
                        xv[pb, j, pl.ds(HI, 16)] + off_hi)
                return carry

            lax.fori_loop(0, NG, grp_body, 0)
            return [
                pltpu.async_copy(
                    w_hbm.at[idx_v.at[pb, g]],
                    gbuf.at[pb, pl.ds(g * GR * F, GR * F), :], sems[pb])
                for g in range(NG)
            ]

        def stage_out(ci, pb, descs):
            for dsc in descs:
                dsc.wait()

            def row_body(j, carry):
                rbase = j * F
                acc = bias_vec
                for f in range(F):
                    acc = acc + gbuf[pb, rbase + f, :]
                outb[j, :] = acc
                return carry

            lax.fori_loop(0, C, row_body, 0)
            base = tbase + ci * C
            pltpu.sync_copy(outb, out_hbm.at[pl.ds(base, C), :])

        descs = stage_in(0, 0)
        for ci in range(NCHUNK):
            nxt = None
            if ci + 1 < NCHUNK:
                nxt = stage_in(ci + 1, (ci + 1) % 2)
            stage_out(ci, ci % 2, descs)
            descs = nxt

    return k(x, weight, off2, bias)


def kernel(x, weight, bias, offset):
    offi = offset.astype(jnp.int32)
    off2 = jnp.stack([offi[0:16], offi[HI:F]])
    return _field_linear_sc(x, weight, off2, bias.astype(jnp.float32))


# speedup vs baseline: 1.0083x; 1.0083x over previous
import functools

import jax
import jax.numpy as jnp
from jax import lax
from jax.experimental import pallas as pl
from jax.experimental.pallas import tpu as pltpu
from jax.experimental.pallas import tpu_sc as plsc

F = 26
OUT = 16
B = 16384
NW = 32
BPT = B // NW
C = 128
NCHUNK = BPT // C
CF = C * F
HI = F - 16


def _build_ids(x, off2):
    mesh = plsc.VectorSubcoreMesh(core_axis_name="c", subcore_axis_name="s")

    @functools.partial(
        pl.kernel,
        out_type=jax.ShapeDtypeStruct((B * F,), jnp.int32),
        mesh=mesh,
        compiler_params=pltpu.CompilerParams(use_tc_tiling_on_sc=True),
        scratch_types=[
            pltpu.VMEM((2, OUT), jnp.int32),
            pltpu.VMEM((BPT, F), jnp.int32),
            pltpu.VMEM((BPT * F,), jnp.int32),
        ],
    )
    def ka(x_hbm, off2_hbm, ids_hbm, off2_v, xv, obuf):
        cid = lax.axis_index("c")
        sid = lax.axis_index("s")
        wid = sid * 2 + cid
        base = wid * BPT
        pltpu.sync_copy(off2_hbm, off2_v)
        pltpu.sync_copy(x_hbm.at[pl.ds(base, BPT), :], xv)
        off_lo = off2_v[0, :]
        off_hi = off2_v[1, :]

        def row_body(j, carry):
            obuf[pl.ds(j * F, 16)] = xv[j, pl.ds(0, 16)] + off_lo
            obuf[pl.ds(j * F + HI, 16)] = xv[j, pl.ds(HI, 16)] + off_hi
            return carry

        lax.fori_loop(0, BPT, row_body, 0)
        pltpu.sync_copy(obuf, ids_hbm.at[pl.ds(base * F, BPT * F)])

    return ka(x, off2)


def _gather_sum(ids, weight, bias):
    mesh = plsc.VectorSubcoreMesh(core_axis_name="c", subcore_axis_name="s")

    @functools.partial(
        pl.kernel,
        out_type=jax.ShapeDtypeStruct((B, OUT), jnp.float32),
        mesh=mesh,
        compiler_params=pltpu.CompilerParams(use_tc_tiling_on_sc=False),
        scratch_types=[
            pltpu.VMEM((OUT,), jnp.float32),
            pltpu.VMEM((2, CF), jnp.int32),
            pltpu.VMEM((2, CF, OUT), jnp.float32),
            pltpu.VMEM((C, OUT), jnp.float32),
            pltpu.SemaphoreType.DMA,
            pltpu.SemaphoreType.DMA,
        ],
    )
    def kb(ids_hbm, w_hbm, bias_hbm, out_hbm,
           bias_v, idx_v, gbuf, outb, sem0, sem1):
        cid = lax.axis_index("c")
        sid = lax.axis_index("s")
        wid = sid * 2 + cid
        tbase = wid * BPT
        sems = (sem0, sem1)

        pltpu.sync_copy(bias_hbm, bias_v)
        bias_vec = bias_v[:]

        def stage_in(ci, pb):
            base = tbase + ci * C
            pltpu.sync_copy(ids_hbm.at[pl.ds(base * F, CF)], idx_v.at[pb])
            return [
                pltpu.async_copy(w_hbm.at[idx_v.at[pb, pl.ds(g * C, C)]],
                                 gbuf.at[pb, pl.ds(g * C, C), :], sems[pb])
                for g in range(F)
            ]

        def stage_out(ci, pb, descs):
            for dsc in descs:
                dsc.wait()

            def row_body(j, carry):
                rbase = j * F
                acc = bias_vec
                for f in range(F):
                    acc = acc + gbuf[pb, rbase + f, :]
                outb[j, :] = acc
                return carry

            lax.fori_loop(0, C, row_body, 0)
            base = tbase + ci * C
            pltpu.sync_copy(outb, out_hbm.at[pl.ds(base, C), :])

        descs = stage_in(0, 0)
        for ci in range(NCHUNK):
            nxt = None
            if ci + 1 < NCHUNK:
                nxt = stage_in(ci + 1, (ci + 1) % 2)
            stage_out(ci, ci % 2, descs)
            descs = nxt

    return kb(ids, weight, bias)


def kernel(x, weight, bias, offset):
    offi = offset.astype(jnp.int32)
    off2 = jnp.stack([offi[0:16], offi[HI:F]])
    ids = _build_ids(x, off2)
    return _gather_sum(ids, weight, bias.astype(jnp.float32))
